# baseline (speedup 1.0000x reference)
import functools

import jax
import jax.numpy as jnp
from jax import lax
from jax.experimental import pallas as pl
from jax.experimental.pallas import tpu as pltpu
from jax.experimental.pallas import tpu_sc as plsc

_D = 64
_NW = 32
_CH = 128


@functools.cache
def _make(B):
    b_per_w = B // _NW
    nch = b_per_w // _CH
    mesh = plsc.VectorSubcoreMesh(core_axis_name="c", subcore_axis_name="s")

    @functools.partial(
        pl.kernel,
        mesh=mesh,
        out_type=jax.ShapeDtypeStruct((B, _D), jnp.float32),
        compiler_params=pltpu.CompilerParams(use_tc_tiling_on_sc=False),
        scratch_types=[
            pltpu.VMEM((nch, _CH), jnp.int32),
            pltpu.VMEM((_CH, _D), jnp.float32),
            pltpu.SemaphoreType.DMA,
        ],
    )
    def emb(idx_hbm, table_hbm, out_hbm, idx_v, rows_v, sem):
        wid = lax.axis_index("s") * 2 + lax.axis_index("c")
        base = wid * b_per_w
        pltpu.sync_copy(idx_hbm.at[wid], idx_v)

        def body(c, carry):
            pltpu.async_copy(table_hbm.at[idx_v.at[c]], rows_v, sem).wait()
            pltpu.sync_copy(rows_v, out_hbm.at[pl.ds(base + c * _CH, _CH)])
            return carry

        lax.fori_loop(0, nch, body, 0)

    return emb


def kernel(x, table):
    n, m = x.shape
    B = n * m
    idx = x.astype(jnp.int32).reshape(_NW, B // _NW // _CH, _CH)
    out = _make(B)(idx, table)
    return out.reshape(n, m, _D)


# speedup vs baseline: 1.0719x; 1.0719x over previous
import functools

import jax
import jax.numpy as jnp
from jax import lax
from jax.experimental import pallas as pl
from jax.experimental.pallas import tpu as pltpu
from jax.experimental.pallas import tpu_sc as plsc

_D = 64
_NW = 32
_GR = 512
_NBUF = 2


@functools.cache
def _make(B):
    b_per_w = B // _NW
    ngrp = b_per_w // _GR
    mesh = plsc.VectorSubcoreMesh(core_axis_name="c", subcore_axis_name="s")

    @functools.partial(
        pl.kernel,
        mesh=mesh,
        out_type=jax.ShapeDtypeStruct((B, _D), jnp.float32),
        compiler_params=pltpu.CompilerParams(use_tc_tiling_on_sc=False),
        scratch_types=[
            *[pltpu.VMEM((_GR,), jnp.int32) for _ in range(_NBUF)],
            *[pltpu.VMEM((_GR, _D), jnp.float32) for _ in range(_NBUF)],
            *[pltpu.SemaphoreType.DMA for _ in range(2 * _NBUF)],
        ],
    )
    def emb(idx_hbm, table_hbm, out_hbm, *refs):
        idxbs = refs[:_NBUF]
        rowss = refs[_NBUF : 2 * _NBUF]
        gsems = refs[2 * _NBUF : 3 * _NBUF]
        osems = refs[3 * _NBUF :]
        wid = lax.axis_index("s") * 2 + lax.axis_index("c")
        base = wid * b_per_w

        def fire(g, b):
            pltpu.sync_copy(idx_hbm.at[wid, g], idxbs[b])
            pltpu.async_copy(table_hbm.at[idxbs[b]], rowss[b], gsems[b])

        def out_slice(g):
            return out_hbm.at[pl.ds(base + g * _GR, _GR)]

        for b in range(_NBUF):
            fire(b, b)

        def outer(i, carry):
            gg = i * _NBUF
            for b in range(_NBUF):
                g = gg + b
                pltpu.make_async_copy(table_hbm.at[idxbs[b]], rowss[b], gsems[b]).wait()
                pltpu.async_copy(rowss[b], out_slice(g), osems[b])

                @pl.when(g + _NBUF < ngrp)
                def _():
                    pltpu.make_async_copy(rowss[b], out_slice(g), osems[b]).wait()
                    fire(g + _NBUF, b)

            return carry

        lax.fori_loop(0, ngrp // _NBUF, outer, 0)

        for b in range(_NBUF):
            g = ngrp - _NBUF + b
            pltpu.make_async_copy(rowss[b], out_slice(g), osems[b]).wait()

    return emb


def kernel(x, table):
    n, m = x.shape
    B = n * m
    idx = x.astype(jnp.int32).reshape(_NW, B // _NW // _GR, _GR)
    out = _make(B)(idx, table)
    return out.reshape(n, m, _D)
